# baseline (speedup 1.0000x reference)
import jax
import jax.numpy as jnp
from jax.experimental import pallas as pl
from jax.experimental.pallas import tpu as pltpu

N = 10000
BM = 400


def _xw1_kernel(x_ref, w1_ref, o_ref):
    o_ref[...] = jnp.dot(x_ref[...], w1_ref[...],
                         preferred_element_type=jnp.float32)


def _pass1_kernel(adj_ref, s1_ref, b1_ref, w2_ref, o_ref):
    h = jnp.dot(adj_ref[...], s1_ref[...],
                preferred_element_type=jnp.float32)
    h = jnp.maximum(h + b1_ref[...], 0.0)
    o_ref[...] = jnp.dot(h, w2_ref[...], preferred_element_type=jnp.float32)


def _pass2_kernel(adj_ref, s2_ref, b2_ref, o_ref):
    o_ref[...] = jnp.dot(adj_ref[...], s2_ref[...],
                         preferred_element_type=jnp.float32) + b2_ref[...]


@jax.jit
def kernel(x, adj, W1, b1, W2, b2):
    nfeat = x.shape[1]
    nhid = W1.shape[1]
    nclass = W2.shape[1]
    b1_2d = b1.reshape(1, nhid)
    b2_2d = b2.reshape(1, nclass)

    s1 = pl.pallas_call(
        _xw1_kernel,
        out_shape=jax.ShapeDtypeStruct((N, nhid), jnp.float32),
        in_specs=[
            pl.BlockSpec((N, nfeat), lambda: (0, 0)),
            pl.BlockSpec((nfeat, nhid), lambda: (0, 0)),
        ],
        out_specs=pl.BlockSpec((N, nhid), lambda: (0, 0)),
    )(x, W1)

    grid = (N // BM,)
    s2 = pl.pallas_call(
        _pass1_kernel,
        grid=grid,
        out_shape=jax.ShapeDtypeStruct((N, nclass), jnp.float32),
        in_specs=[
            pl.BlockSpec((BM, N), lambda i: (i, 0)),
            pl.BlockSpec((N, nhid), lambda i: (0, 0)),
            pl.BlockSpec((1, nhid), lambda i: (0, 0)),
            pl.BlockSpec((nhid, nclass), lambda i: (0, 0)),
        ],
        out_specs=pl.BlockSpec((BM, nclass), lambda i: (i, 0)),
        compiler_params=pltpu.CompilerParams(
            dimension_semantics=("arbitrary",),
        ),
    )(adj, s1, b1_2d, W2)

    out = pl.pallas_call(
        _pass2_kernel,
        grid=grid,
        out_shape=jax.ShapeDtypeStruct((N, nclass), jnp.float32),
        in_specs=[
            pl.BlockSpec((BM, N), lambda i: (i, 0)),
            pl.BlockSpec((N, nclass), lambda i: (0, 0)),
            pl.BlockSpec((1, nclass), lambda i: (0, 0)),
        ],
        out_specs=pl.BlockSpec((BM, nclass), lambda i: (i, 0)),
        compiler_params=pltpu.CompilerParams(
            dimension_semantics=("arbitrary",),
        ),
    )(adj, s2, b2_2d)

    return out


# speedup vs baseline: 1.0527x; 1.0527x over previous
import jax
import jax.numpy as jnp
from jax.experimental import pallas as pl
from jax.experimental.pallas import tpu as pltpu

N = 10000
BM = 400
NB = N // BM


def _gcn_kernel(adj_ref, x_ref, w1_ref, b1_ref, w2_ref, b2_ref,
                out_ref, s1_scr, s2_scr):
    i = pl.program_id(0)

    @pl.when(i == 0)
    def _():
        s1_scr[...] = jnp.dot(x_ref[...], w1_ref[...],
                              preferred_element_type=jnp.float32)

    @pl.when(i < NB)
    def _():
        h = jnp.dot(adj_ref[...], s1_scr[...],
                    preferred_element_type=jnp.float32)
        h = jnp.maximum(h + b1_ref[...], 0.0)
        s2_scr[pl.ds(i * BM, BM), :] = jnp.dot(
            h, w2_ref[...], preferred_element_type=jnp.float32)

    @pl.when(i >= NB)
    def _():
        out_ref[...] = jnp.dot(adj_ref[...], s2_scr[...],
                               preferred_element_type=jnp.float32) + b2_ref[...]


@jax.jit
def kernel(x, adj, W1, b1, W2, b2):
    nfeat = x.shape[1]
    nhid = W1.shape[1]
    nclass = W2.shape[1]
    b1_2d = b1.reshape(1, nhid)
    b2_2d = b2.reshape(1, nclass)

    return pl.pallas_call(
        _gcn_kernel,
        grid=(2 * NB,),
        out_shape=jax.ShapeDtypeStruct((N, nclass), jnp.float32),
        in_specs=[
            pl.BlockSpec((BM, N), lambda i: (jax.lax.rem(i, NB), 0)),
            pl.BlockSpec((N, nfeat), lambda i: (0, 0)),
            pl.BlockSpec((nfeat, nhid), lambda i: (0, 0)),
            pl.BlockSpec((1, nhid), lambda i: (0, 0)),
            pl.BlockSpec((nhid, nclass), lambda i: (0, 0)),
            pl.BlockSpec((1, nclass), lambda i: (0, 0)),
        ],
        out_specs=pl.BlockSpec(
            (BM, nclass),
            lambda i: (jnp.where(i < NB, 0, i - NB), 0)),
        scratch_shapes=[
            pltpu.VMEM((N, nhid), jnp.float32),
            pltpu.VMEM((N, nclass), jnp.float32),
        ],
        compiler_params=pltpu.CompilerParams(
            dimension_semantics=("arbitrary",),
        ),
    )(adj, x, W1, b1_2d, W2, b2_2d)
